# baseline (speedup 1.0000x reference)
import functools

import jax
import jax.numpy as jnp
from jax import lax
from jax.experimental import pallas as pl
from jax.experimental.pallas import tpu as pltpu
from jax.experimental.pallas import tpu_sc as plsc

HLEN, WLEN = 128, 128
CHUNK = 128


def _crop_body(hw_ref, bg_ref, out_ref):
    h = hw_ref[0]
    w = hw_ref[1]
    val = bg_ref[0, 0]
    val = pltpu.roll(val, -h, 0)
    val = pltpu.roll(val, -w, 1)
    out_ref[0, 0] = val[:HLEN, :WLEN]


def _make_sc_gather(n_rows_out, n_rows_table):
    info = plsc.get_sparse_core_info()
    nc, ns = info.num_cores, info.num_subcores
    nw = nc * ns
    rpw = n_rows_out // nw
    nchunks = rpw // CHUNK
    mesh = plsc.VectorSubcoreMesh(core_axis_name="c", subcore_axis_name="s")

    @functools.partial(
        pl.kernel,
        out_type=jax.ShapeDtypeStruct((n_rows_out, WLEN), jnp.float32),
        mesh=mesh,
        scratch_types=[
            pltpu.VMEM((rpw,), jnp.int32),
            pltpu.VMEM((CHUNK, WLEN), jnp.float32),
            pltpu.VMEM((CHUNK, WLEN), jnp.float32),
            pltpu.SemaphoreType.DMA,
            pltpu.SemaphoreType.DMA,
            pltpu.SemaphoreType.DMA,
            pltpu.SemaphoreType.DMA,
        ],
    )
    def sc_gather(table_hbm, rowidx_hbm, out_hbm, idx_v, b0, b1, g0, g1, s0, s1):
        wid = lax.axis_index("s") * nc + lax.axis_index("c")
        base = wid * rpw
        bufs, gsems, ssems = (b0, b1), (g0, g1), (s0, s1)

        pltpu.sync_copy(rowidx_hbm.at[pl.ds(base, rpw)], idx_v)

        def gcopy(k, p):
            return pltpu.make_async_copy(
                table_hbm.at[idx_v.at[pl.ds(k * CHUNK, CHUNK)]], bufs[p], gsems[p]
            )

        def scopy(k, p):
            return pltpu.make_async_copy(
                bufs[p], out_hbm.at[pl.ds(base + k * CHUNK, CHUNK)], ssems[p]
            )

        for k in range(nchunks):
            p = k % 2
            if k >= 2:
                scopy(k - 2, p).wait()
            gcopy(k, p).start()
            if k >= 1:
                q = 1 - p
                gcopy(k - 1, q).wait()
                scopy(k - 1, q).start()
        pl_ = (nchunks - 1) % 2
        gcopy(nchunks - 1, pl_).wait()
        scopy(nchunks - 1, pl_).start()
        scopy(nchunks - 2, 1 - pl_).wait()
        scopy(nchunks - 1, pl_).wait()

    return sc_gather


def kernel(background, image_id_indices, h, w):
    n_img, c, height, width = background.shape
    batch = image_id_indices.shape[0]
    rows_per_b = c * HLEN

    hw = jnp.stack([jnp.asarray(h, jnp.int32), jnp.asarray(w, jnp.int32)])

    crop = pl.pallas_call(
        _crop_body,
        grid_spec=pltpu.PrefetchScalarGridSpec(
            num_scalar_prefetch=1,
            grid=(n_img, c),
            in_specs=[
                pl.BlockSpec((1, 1, height, width), lambda i, j, hw_ref: (i, j, 0, 0)),
            ],
            out_specs=pl.BlockSpec((1, 1, HLEN, WLEN), lambda i, j, hw_ref: (i, j, 0, 0)),
        ),
        out_shape=jax.ShapeDtypeStruct((n_img, c, HLEN, WLEN), background.dtype),
    )
    table = crop(hw, background)

    rowidx = (
        image_id_indices[:, None] * rows_per_b + jnp.arange(rows_per_b, dtype=jnp.int32)
    ).reshape(-1)

    sc_gather = _make_sc_gather(batch * rows_per_b, n_img * rows_per_b)
    out = sc_gather(table.reshape(n_img * rows_per_b, WLEN), rowidx)
    return out.reshape(batch, c, HLEN, WLEN)
